# baseline (speedup 1.0000x reference)
import jax
import jax.numpy as jnp
from jax import lax
from jax.experimental import pallas as pl
from jax.experimental.pallas import tpu as pltpu
from jax.experimental.pallas import tpu_sc as plsc

_NUM_FIELDS = 26
_VOCAB = 100000
_EMB_DIM = 50
_EMB_PAD = 56
_BATCH = 16384

_NC = 2
_NS = 16
_NW = _NC * _NS

_ROWS = _BATCH * _NUM_FIELDS
_PER_W = _ROWS // _NW
_CH = 128
_K = _PER_W // _CH
_NBUF = 4


def _gather_body(tab_hbm, idx_hbm, out_hbm, idx_v, rows_v, *sems):
    wid = lax.axis_index("s") * _NC + lax.axis_index("c")
    base = wid * _PER_W
    pltpu.sync_copy(idx_hbm.at[wid], idx_v)
    for b in range(_NBUF):
        pltpu.async_copy(tab_hbm.at[idx_v.at[b]], rows_v.at[b], sems[b])

    @pl.loop(0, _K, step=_NBUF)
    def _(j0):
        for b in range(_NBUF):
            j = j0 + b
            pltpu.make_async_copy(
                tab_hbm.at[idx_v.at[b]], rows_v.at[b], sems[b]
            ).wait()
            pltpu.sync_copy(rows_v.at[b], out_hbm.at[pl.ds(base + j * _CH, _CH)])
            nj = j + _NBUF

            @pl.when(nj < _K)
            def _():
                pltpu.async_copy(tab_hbm.at[idx_v.at[nj]], rows_v.at[b], sems[b])


@jax.jit
def kernel(x_cat, tables):
    offs = (jnp.arange(_NUM_FIELDS, dtype=jnp.int32) * _VOCAB)[None, :]
    gidx = (x_cat.astype(jnp.int32) + offs).reshape(_NW, _K, _CH)
    tab = tables.reshape(_NUM_FIELDS * _VOCAB, _EMB_DIM)
    tab = jnp.pad(tab, ((0, 0), (0, _EMB_PAD - _EMB_DIM)))

    mesh = plsc.VectorSubcoreMesh(core_axis_name="c", subcore_axis_name="s")
    out = pl.kernel(
        _gather_body,
        out_type=jax.ShapeDtypeStruct((_ROWS, _EMB_PAD), jnp.float32),
        mesh=mesh,
        scratch_types=[
            pltpu.VMEM((_K, _CH), jnp.int32),
            pltpu.VMEM((_NBUF, _CH, _EMB_PAD), jnp.float32),
        ] + [pltpu.SemaphoreType.DMA] * _NBUF,
        compiler_params=pltpu.CompilerParams(use_tc_tiling_on_sc=False),
    )(tab, gidx)
    return out[:, :_EMB_DIM].reshape(_BATCH, _NUM_FIELDS * _EMB_DIM)
